# baseline (speedup 1.0000x reference)
import functools

import jax
import jax.numpy as jnp
from jax import lax
from jax.experimental import pallas as pl
from jax.experimental.pallas import tpu as pltpu
from jax.experimental.pallas import tpu_sc as plsc

N = 10000
D = 128
E = 320000
EPS = 1e-5

NC = 2
NS = 16
NW = NC * NS
NP = 10240
EP = 327680
EPT = EP // NW
CH = 128
TCH = EP // CH
K0 = 160
SL = 40
DUMMY = 10100
RB = 1280
G = NP // RB

_mesh = plsc.VectorSubcoreMesh(core_axis_name="c", subcore_axis_name="s")


@functools.partial(
    pl.kernel,
    out_type=jax.ShapeDtypeStruct((NW, NP), jnp.float32),
    mesh=_mesh,
    compiler_params=pltpu.CompilerParams(needs_layout_passes=False),
    scratch_types=[
        pltpu.VMEM((EPT,), jnp.int32),
        pltpu.VMEM((NP,), jnp.float32),
    ],
)
def _deg_kernel(col_hbm, out_hbm, col_v, deg_v):
    wid = lax.axis_index("s") * NC + lax.axis_index("c")
    pltpu.sync_copy(col_hbm.at[wid], col_v)
    zeros16 = jnp.zeros((16,), jnp.float32)

    def zbody(i, _):
        deg_v[pl.ds(i * 16, 16)] = zeros16
        return 0

    lax.fori_loop(0, NP // 16, zbody, 0)
    ones16 = jnp.ones((16,), jnp.float32)

    def hbody(i, _):
        idx = col_v[pl.ds(i * 16, 16)]
        plsc.addupdate_scatter(deg_v, [idx], ones16)
        return 0

    lax.fori_loop(0, EPT // 16, hbody, 0)
    pltpu.sync_copy(deg_v, out_hbm.at[wid])


@functools.partial(
    pl.kernel,
    out_type=jax.ShapeDtypeStruct((NP, D), jnp.float32),
    mesh=_mesh,
    compiler_params=pltpu.CompilerParams(needs_layout_passes=False),
    scratch_types=[
        pltpu.VMEM((SL, CH), jnp.int32),
        pltpu.VMEM((SL, CH), jnp.int32),
        pltpu.VMEM((CH, D), jnp.float32),
        pltpu.VMEM((CH, D), jnp.float32),
        pltpu.VMEM_SHARED((NP, D), jnp.float32),
        pltpu.SemaphoreType.DMA,
        pltpu.SemaphoreType.DMA,
        pltpu.SemaphoreType.DMA,
        pltpu.SemaphoreType.DMA,
    ],
)
def _agg_kernel(y_hbm, row_hbm, col_hbm, out_hbm,
                row_v, col_v, buf0, buf1, acc, gsem0, gsem1, ssem0, ssem1):
    cid = lax.axis_index("c")
    sid = lax.axis_index("s")
    rpt = NP // NS

    @pl.when(cid == 0)
    def _core0():
        zeros16 = jnp.zeros((16,), jnp.float32)

        def zbody(i, _):
            for k in range(D // 16):
                buf0[i, pl.ds(k * 16, 16)] = zeros16
            return 0

        lax.fori_loop(0, CH, zbody, 0)
        for k in range(rpt // CH):
            pltpu.sync_copy(buf0, acc.at[pl.ds(sid * rpt + k * CH, CH)])
        plsc.subcore_barrier()

        bufs = (buf0, buf1)
        gsems = (gsem0, gsem1)
        ssems = (ssem0, ssem1)

        for slab in range(K0 // SL):
            cb = sid * K0 + slab * SL
            pltpu.sync_copy(row_hbm.at[pl.ds(cb, SL)], row_v)
            pltpu.sync_copy(col_hbm.at[pl.ds(cb, SL)], col_v)
            pltpu.async_copy(y_hbm.at[row_v.at[0]], buf0, gsem0)

            def body(i, _):
                for t in (0, 1):
                    j = 2 * i + t
                    pltpu.make_async_copy(y_hbm.at[row_v.at[j]], bufs[t],
                                          gsems[t]).wait()
                    pltpu.async_copy(bufs[t], acc.at[col_v.at[j]],
                                     ssems[t], add=True)

                    @pl.when(j >= 1)
                    def _drain_prev():
                        pltpu.make_async_copy(
                            bufs[1 - t], acc.at[col_v.at[j - 1]],
                            ssems[1 - t]).wait()

                    @pl.when(j + 1 < SL)
                    def _start_next():
                        pltpu.async_copy(y_hbm.at[row_v.at[j + 1]],
                                         bufs[1 - t], gsems[1 - t])
                return 0

            lax.fori_loop(0, SL // 2, body, 0)
            pltpu.make_async_copy(bufs[1], acc.at[col_v.at[SL - 1]],
                                  ssems[1]).wait()

        plsc.subcore_barrier()
        pltpu.sync_copy(acc.at[pl.ds(sid * rpt, rpt)],
                        out_hbm.at[pl.ds(sid * rpt, rpt)])


def _dis_body(part_ref, o_ref):
    s = jnp.sum(part_ref[...], axis=0) + 1.0
    o_ref[...] = lax.rsqrt(s)


_dis_call = pl.pallas_call(
    _dis_body,
    out_shape=jax.ShapeDtypeStruct((NP,), jnp.float32),
)


def _scale_body(x_ref, dis_ref, o_ref):
    o_ref[...] = x_ref[...] * dis_ref[...]


_scale_call = pl.pallas_call(
    _scale_body,
    grid=(G,),
    in_specs=[
        pl.BlockSpec((RB, D), lambda i: (i, 0)),
        pl.BlockSpec((RB, 1), lambda i: (i, 0)),
    ],
    out_specs=pl.BlockSpec((RB, D), lambda i: (i, 0)),
    out_shape=jax.ShapeDtypeStruct((NP, D), jnp.float32),
)


def _mm_body(p_ref, y_ref, dis_ref, w_ref, b_ref, h_ref, mom_ref):
    i = pl.program_id(0)
    z = (p_ref[...] + y_ref[...]) * dis_ref[...]
    h = jnp.dot(z, w_ref[...], preferred_element_type=jnp.float32) + b_ref[...]
    h_ref[...] = h
    rows = i * RB + lax.broadcasted_iota(jnp.int32, (RB, 1), 0)
    hm = jnp.where(rows < N, h, 0.0)
    s1 = jnp.sum(hm, axis=0, keepdims=True)
    s2 = jnp.sum(hm * hm, axis=0, keepdims=True)
    mom_ref[...] = jnp.concatenate([s1, s2], axis=0).reshape(1, 2, D)


_mm_call = pl.pallas_call(
    _mm_body,
    grid=(G,),
    in_specs=[
        pl.BlockSpec((RB, D), lambda i: (i, 0)),
        pl.BlockSpec((RB, D), lambda i: (i, 0)),
        pl.BlockSpec((RB, 1), lambda i: (i, 0)),
        pl.BlockSpec((D, D), lambda i: (0, 0)),
        pl.BlockSpec((1, D), lambda i: (0, 0)),
    ],
    out_specs=[
        pl.BlockSpec((RB, D), lambda i: (i, 0)),
        pl.BlockSpec((1, 2, D), lambda i: (i, 0, 0)),
    ],
    out_shape=[
        jax.ShapeDtypeStruct((NP, D), jnp.float32),
        jax.ShapeDtypeStruct((G, 2, D), jnp.float32),
    ],
)


def _bn_body(h_ref, mom_ref, g_ref, bt_ref, o_ref):
    m = jnp.sum(mom_ref[...], axis=0)
    mean = m[0:1] * (1.0 / N)
    var = m[1:2] * (1.0 / N) - mean * mean
    scale = g_ref[...] * lax.rsqrt(var + EPS)
    shift = bt_ref[...] - mean * scale
    o_ref[...] = jnp.maximum(h_ref[...] * scale + shift, 0.0)


_bn_call = pl.pallas_call(
    _bn_body,
    grid=(G,),
    in_specs=[
        pl.BlockSpec((RB, D), lambda i: (i, 0)),
        pl.BlockSpec((G, 2, D), lambda i: (0, 0, 0)),
        pl.BlockSpec((1, D), lambda i: (0, 0)),
        pl.BlockSpec((1, D), lambda i: (0, 0)),
    ],
    out_specs=pl.BlockSpec((RB, D), lambda i: (i, 0)),
    out_shape=jax.ShapeDtypeStruct((NP, D), jnp.float32),
)


def kernel(x, edge_index, W, b, gamma, beta):
    x = x.astype(jnp.float32)
    ei = edge_index.astype(jnp.int32)
    rowp = jnp.concatenate([ei[0], jnp.zeros((EP - E,), jnp.int32)])
    colp = jnp.concatenate([ei[1], jnp.full((EP - E,), DUMMY, jnp.int32)])
    col_flat = colp.reshape(NW, EPT)
    row3 = rowp.reshape(TCH, CH)
    col3 = colp.reshape(TCH, CH)

    parts = _deg_kernel(col_flat)
    dis = _dis_call(parts)
    dis_col = dis.reshape(NP, 1)
    x_pad = jnp.concatenate([x, jnp.zeros((NP - N, D), jnp.float32)])
    y = _scale_call(x_pad, dis_col)
    p = _agg_kernel(y, row3, col3)
    h, mom = _mm_call(p, y, dis_col,
                      W.astype(jnp.float32),
                      b.astype(jnp.float32).reshape(1, D))
    out = _bn_call(h, mom,
                   gamma.astype(jnp.float32).reshape(1, D),
                   beta.astype(jnp.float32).reshape(1, D))
    return out[:N]


# speedup vs baseline: 1.0012x; 1.0012x over previous
import functools

import jax
import jax.numpy as jnp
from jax import lax
from jax.experimental import pallas as pl
from jax.experimental.pallas import tpu as pltpu
from jax.experimental.pallas import tpu_sc as plsc

N = 10000
D = 128
E = 320000
EPS = 1e-5

NC = 2
NS = 16
NW = NC * NS
NP = 10240
EP = 327680
EPT = EP // NW
CH = 128
TCH = EP // CH
K0 = 160
SL = 40
DUMMY = 10100
RB = 1280
G = NP // RB

_mesh = plsc.VectorSubcoreMesh(core_axis_name="c", subcore_axis_name="s")


@functools.partial(
    pl.kernel,
    out_type=jax.ShapeDtypeStruct((NW, NP), jnp.float32),
    mesh=_mesh,
    compiler_params=pltpu.CompilerParams(needs_layout_passes=False),
    scratch_types=[
        pltpu.VMEM((EPT,), jnp.int32),
        pltpu.VMEM((NP,), jnp.float32),
    ],
)
def _deg_kernel(col_hbm, out_hbm, col_v, deg_v):
    wid = lax.axis_index("s") * NC + lax.axis_index("c")
    pltpu.sync_copy(col_hbm.at[wid], col_v)
    zeros16 = jnp.zeros((16,), jnp.float32)

    def zbody(i, _):
        deg_v[pl.ds(i * 16, 16)] = zeros16
        return 0

    lax.fori_loop(0, NP // 16, zbody, 0)
    ones16 = jnp.ones((16,), jnp.float32)

    def hbody(i, _):
        idx = col_v[pl.ds(i * 16, 16)]
        plsc.addupdate_scatter(deg_v, [idx], ones16)
        return 0

    lax.fori_loop(0, EPT // 16, hbody, 0)
    pltpu.sync_copy(deg_v, out_hbm.at[wid])


@functools.partial(
    pl.kernel,
    out_type=jax.ShapeDtypeStruct((NP, D), jnp.float32),
    mesh=_mesh,
    compiler_params=pltpu.CompilerParams(needs_layout_passes=False),
    scratch_types=[
        pltpu.VMEM((SL, CH), jnp.int32),
        pltpu.VMEM((SL, CH), jnp.int32),
        pltpu.VMEM((CH, D), jnp.float32),
        pltpu.VMEM((CH, D), jnp.float32),
        pltpu.VMEM_SHARED((NP, D), jnp.float32),
        pltpu.SemaphoreType.DMA,
        pltpu.SemaphoreType.DMA,
        pltpu.SemaphoreType.DMA,
        pltpu.SemaphoreType.DMA,
    ],
)
def _agg_kernel(y_hbm, row_hbm, col_hbm, out_hbm,
                row_v, col_v, buf0, buf1, acc, gsem0, gsem1, ssem0, ssem1):
    cid = lax.axis_index("c")
    sid = lax.axis_index("s")
    rpt = NP // NS

    @pl.when(cid == 0)
    def _core0():
        zeros16 = jnp.zeros((16,), jnp.float32)

        def zbody(i, _):
            for k in range(D // 16):
                buf0[i, pl.ds(k * 16, 16)] = zeros16
            return 0

        lax.fori_loop(0, CH, zbody, 0)
        for k in range(rpt // CH):
            pltpu.sync_copy(buf0, acc.at[pl.ds(sid * rpt + k * CH, CH)])
        plsc.subcore_barrier()

        bufs = (buf0, buf1)
        gsems = (gsem0, gsem1)
        ssems = (ssem0, ssem1)

        for slab in range(K0 // SL):
            cb = sid * K0 + slab * SL
            pltpu.sync_copy(row_hbm.at[pl.ds(cb, SL)], row_v)
            pltpu.sync_copy(col_hbm.at[pl.ds(cb, SL)], col_v)
            pltpu.async_copy(y_hbm.at[row_v.at[0]], buf0, gsem0)

            def body(i, _):
                for t in (0, 1):
                    j = 2 * i + t
                    pltpu.make_async_copy(y_hbm.at[row_v.at[j]], bufs[t],
                                          gsems[t]).wait()

                    @pl.when(j + 1 < SL)
                    def _start_next():
                        pltpu.async_copy(y_hbm.at[row_v.at[j + 1]],
                                         bufs[1 - t], gsems[1 - t])

                    pltpu.sync_copy(bufs[t], acc.at[col_v.at[j]], add=True)
                return 0

            lax.fori_loop(0, SL // 2, body, 0)

        plsc.subcore_barrier()
        pltpu.sync_copy(acc.at[pl.ds(sid * rpt, rpt)],
                        out_hbm.at[pl.ds(sid * rpt, rpt)])


def _dis_body(part_ref, o_ref):
    s = jnp.sum(part_ref[...], axis=0) + 1.0
    o_ref[...] = lax.rsqrt(s)


_dis_call = pl.pallas_call(
    _dis_body,
    out_shape=jax.ShapeDtypeStruct((NP,), jnp.float32),
)


def _scale_body(x_ref, dis_ref, o_ref):
    o_ref[...] = x_ref[...] * dis_ref[...]


_scale_call = pl.pallas_call(
    _scale_body,
    grid=(G,),
    in_specs=[
        pl.BlockSpec((RB, D), lambda i: (i, 0)),
        pl.BlockSpec((RB, 1), lambda i: (i, 0)),
    ],
    out_specs=pl.BlockSpec((RB, D), lambda i: (i, 0)),
    out_shape=jax.ShapeDtypeStruct((NP, D), jnp.float32),
)


def _mm_body(p_ref, y_ref, dis_ref, w_ref, b_ref, h_ref, mom_ref):
    i = pl.program_id(0)
    z = (p_ref[...] + y_ref[...]) * dis_ref[...]
    h = jnp.dot(z, w_ref[...], preferred_element_type=jnp.float32) + b_ref[...]
    h_ref[...] = h
    rows = i * RB + lax.broadcasted_iota(jnp.int32, (RB, 1), 0)
    hm = jnp.where(rows < N, h, 0.0)
    s1 = jnp.sum(hm, axis=0, keepdims=True)
    s2 = jnp.sum(hm * hm, axis=0, keepdims=True)
    mom_ref[...] = jnp.concatenate([s1, s2], axis=0).reshape(1, 2, D)


_mm_call = pl.pallas_call(
    _mm_body,
    grid=(G,),
    in_specs=[
        pl.BlockSpec((RB, D), lambda i: (i, 0)),
        pl.BlockSpec((RB, D), lambda i: (i, 0)),
        pl.BlockSpec((RB, 1), lambda i: (i, 0)),
        pl.BlockSpec((D, D), lambda i: (0, 0)),
        pl.BlockSpec((1, D), lambda i: (0, 0)),
    ],
    out_specs=[
        pl.BlockSpec((RB, D), lambda i: (i, 0)),
        pl.BlockSpec((1, 2, D), lambda i: (i, 0, 0)),
    ],
    out_shape=[
        jax.ShapeDtypeStruct((NP, D), jnp.float32),
        jax.ShapeDtypeStruct((G, 2, D), jnp.float32),
    ],
)


def _bn_body(h_ref, mom_ref, g_ref, bt_ref, o_ref):
    m = jnp.sum(mom_ref[...], axis=0)
    mean = m[0:1] * (1.0 / N)
    var = m[1:2] * (1.0 / N) - mean * mean
    scale = g_ref[...] * lax.rsqrt(var + EPS)
    shift = bt_ref[...] - mean * scale
    o_ref[...] = jnp.maximum(h_ref[...] * scale + shift, 0.0)


_bn_call = pl.pallas_call(
    _bn_body,
    grid=(G,),
    in_specs=[
        pl.BlockSpec((RB, D), lambda i: (i, 0)),
        pl.BlockSpec((G, 2, D), lambda i: (0, 0, 0)),
        pl.BlockSpec((1, D), lambda i: (0, 0)),
        pl.BlockSpec((1, D), lambda i: (0, 0)),
    ],
    out_specs=pl.BlockSpec((RB, D), lambda i: (i, 0)),
    out_shape=jax.ShapeDtypeStruct((NP, D), jnp.float32),
)


def kernel(x, edge_index, W, b, gamma, beta):
    x = x.astype(jnp.float32)
    ei = edge_index.astype(jnp.int32)
    rowp = jnp.concatenate([ei[0], jnp.zeros((EP - E,), jnp.int32)])
    colp = jnp.concatenate([ei[1], jnp.full((EP - E,), DUMMY, jnp.int32)])
    col_flat = colp.reshape(NW, EPT)
    row3 = rowp.reshape(TCH, CH)
    col3 = colp.reshape(TCH, CH)

    parts = _deg_kernel(col_flat)
    dis = _dis_call(parts)
    dis_col = dis.reshape(NP, 1)
    x_pad = jnp.concatenate([x, jnp.zeros((NP - N, D), jnp.float32)])
    y = _scale_call(x_pad, dis_col)
    p = _agg_kernel(y, row3, col3)
    h, mom = _mm_call(p, y, dis_col,
                      W.astype(jnp.float32),
                      b.astype(jnp.float32).reshape(1, D))
    out = _bn_call(h, mom,
                   gamma.astype(jnp.float32).reshape(1, D),
                   beta.astype(jnp.float32).reshape(1, D))
    return out[:N]


# speedup vs baseline: 1.3499x; 1.3484x over previous
import functools

import jax
import jax.numpy as jnp
from jax import lax
from jax.experimental import pallas as pl
from jax.experimental.pallas import tpu as pltpu
from jax.experimental.pallas import tpu_sc as plsc

N = 10000
D = 128
E = 320000
EPS = 1e-5

NC = 2
NS = 16
NW = NC * NS
NP = 10240
EP = 327680
EPT = EP // NW
CH = 128
TCH = EP // CH
K0 = 120
K1 = 40
SL = 40
DUMMY = 10100
RB = 1280
G = NP // RB

_mesh = plsc.VectorSubcoreMesh(core_axis_name="c", subcore_axis_name="s")


@functools.partial(
    pl.kernel,
    out_type=jax.ShapeDtypeStruct((NW, NP), jnp.float32),
    mesh=_mesh,
    compiler_params=pltpu.CompilerParams(needs_layout_passes=False),
    scratch_types=[
        pltpu.VMEM((EPT,), jnp.int32),
        pltpu.VMEM((NP,), jnp.float32),
    ],
)
def _deg_kernel(col_hbm, out_hbm, col_v, deg_v):
    wid = lax.axis_index("s") * NC + lax.axis_index("c")
    pltpu.sync_copy(col_hbm.at[wid], col_v)
    zeros16 = jnp.zeros((16,), jnp.float32)

    def zbody(i, _):
        deg_v[pl.ds(i * 16, 16)] = zeros16
        return 0

    lax.fori_loop(0, NP // 16, zbody, 0)
    ones16 = jnp.ones((16,), jnp.float32)

    def hbody(i, _):
        idx = col_v[pl.ds(i * 16, 16)]
        plsc.addupdate_scatter(deg_v, [idx], ones16)
        return 0

    lax.fori_loop(0, EPT // 16, hbody, 0)
    pltpu.sync_copy(deg_v, out_hbm.at[wid])


@functools.partial(
    pl.kernel,
    out_type=jax.ShapeDtypeStruct((NC, NP, D), jnp.float32),
    mesh=_mesh,
    compiler_params=pltpu.CompilerParams(needs_layout_passes=False),
    scratch_types=[
        pltpu.VMEM((SL, CH), jnp.int32),
        pltpu.VMEM((SL, CH), jnp.int32),
        pltpu.VMEM((CH, D), jnp.float32),
        pltpu.VMEM((CH, D), jnp.float32),
        pltpu.VMEM_SHARED((NP, D), jnp.float32),
        pltpu.SemaphoreType.DMA,
        pltpu.SemaphoreType.DMA,
    ],
)
def _agg_kernel(y_hbm, row_hbm, col_hbm, out_hbm,
                row_v, col_v, buf0, buf1, acc, sem0, sem1):
    cid = lax.axis_index("c")
    sid = lax.axis_index("s")
    rpt = NP // NS

    zeros16 = jnp.zeros((16,), jnp.float32)

    def zbody(i, _):
        for k in range(D // 16):
            buf0[i, pl.ds(k * 16, 16)] = zeros16
        return 0

    lax.fori_loop(0, CH, zbody, 0)
    for k in range(rpt // CH):
        pltpu.sync_copy(buf0, acc.at[pl.ds(sid * rpt + k * CH, CH)])
    plsc.subcore_barrier()

    bufs = (buf0, buf1)
    sems = (sem0, sem1)

    def run_side(base, k):
        for slab in range(k // SL):
            cb = base + slab * SL
            pltpu.sync_copy(row_hbm.at[pl.ds(cb, SL)], row_v)
            pltpu.sync_copy(col_hbm.at[pl.ds(cb, SL)], col_v)
            pltpu.async_copy(y_hbm.at[row_v.at[0]], buf0, sem0)

            def body(i, _):
                for t in (0, 1):
                    j = 2 * i + t
                    pltpu.make_async_copy(y_hbm.at[row_v.at[j]], bufs[t],
                                          sems[t]).wait()

                    @pl.when(j + 1 < SL)
                    def _start_next():
                        pltpu.async_copy(y_hbm.at[row_v.at[j + 1]],
                                         bufs[1 - t], sems[1 - t])

                    pltpu.sync_copy(bufs[t], acc.at[col_v.at[j]], add=True)
                return 0

            lax.fori_loop(0, SL // 2, body, 0)

    @pl.when(cid == 0)
    def _side0():
        run_side(sid * K0, K0)

    @pl.when(cid == 1)
    def _side1():
        run_side(NS * K0 + sid * K1, K1)

    plsc.subcore_barrier()
    pltpu.sync_copy(acc.at[pl.ds(sid * rpt, rpt)],
                    out_hbm.at[cid, pl.ds(sid * rpt, rpt)])


def _dis_body(part_ref, o_ref):
    s = jnp.sum(part_ref[...], axis=0) + 1.0
    o_ref[...] = lax.rsqrt(s)


_dis_call = pl.pallas_call(
    _dis_body,
    out_shape=jax.ShapeDtypeStruct((NP,), jnp.float32),
)


def _scale_body(x_ref, dis_ref, o_ref):
    o_ref[...] = x_ref[...] * dis_ref[...]


_scale_call = pl.pallas_call(
    _scale_body,
    grid=(G,),
    in_specs=[
        pl.BlockSpec((RB, D), lambda i: (i, 0)),
        pl.BlockSpec((RB, 1), lambda i: (i, 0)),
    ],
    out_specs=pl.BlockSpec((RB, D), lambda i: (i, 0)),
    out_shape=jax.ShapeDtypeStruct((NP, D), jnp.float32),
)


def _mm_body(p_ref, y_ref, dis_ref, w_ref, b_ref, h_ref, mom_ref):
    i = pl.program_id(0)
    z = (p_ref[0] + p_ref[1] + y_ref[...]) * dis_ref[...]
    h = jnp.dot(z, w_ref[...], preferred_element_type=jnp.float32) + b_ref[...]
    h_ref[...] = h
    rows = i * RB + lax.broadcasted_iota(jnp.int32, (RB, 1), 0)
    hm = jnp.where(rows < N, h, 0.0)
    s1 = jnp.sum(hm, axis=0, keepdims=True)
    s2 = jnp.sum(hm * hm, axis=0, keepdims=True)
    mom_ref[...] = jnp.concatenate([s1, s2], axis=0).reshape(1, 2, D)


_mm_call = pl.pallas_call(
    _mm_body,
    grid=(G,),
    in_specs=[
        pl.BlockSpec((NC, RB, D), lambda i: (0, i, 0)),
        pl.BlockSpec((RB, D), lambda i: (i, 0)),
        pl.BlockSpec((RB, 1), lambda i: (i, 0)),
        pl.BlockSpec((D, D), lambda i: (0, 0)),
        pl.BlockSpec((1, D), lambda i: (0, 0)),
    ],
    out_specs=[
        pl.BlockSpec((RB, D), lambda i: (i, 0)),
        pl.BlockSpec((1, 2, D), lambda i: (i, 0, 0)),
    ],
    out_shape=[
        jax.ShapeDtypeStruct((NP, D), jnp.float32),
        jax.ShapeDtypeStruct((G, 2, D), jnp.float32),
    ],
)


def _bn_body(h_ref, mom_ref, g_ref, bt_ref, o_ref):
    m = jnp.sum(mom_ref[...], axis=0)
    mean = m[0:1] * (1.0 / N)
    var = m[1:2] * (1.0 / N) - mean * mean
    scale = g_ref[...] * lax.rsqrt(var + EPS)
    shift = bt_ref[...] - mean * scale
    o_ref[...] = jnp.maximum(h_ref[...] * scale + shift, 0.0)


_BNB = 1000

_bn_call = pl.pallas_call(
    _bn_body,
    grid=(N // _BNB,),
    in_specs=[
        pl.BlockSpec((_BNB, D), lambda i: (i, 0)),
        pl.BlockSpec((G, 2, D), lambda i: (0, 0, 0)),
        pl.BlockSpec((1, D), lambda i: (0, 0)),
        pl.BlockSpec((1, D), lambda i: (0, 0)),
    ],
    out_specs=pl.BlockSpec((_BNB, D), lambda i: (i, 0)),
    out_shape=jax.ShapeDtypeStruct((N, D), jnp.float32),
)


def kernel(x, edge_index, W, b, gamma, beta):
    x = x.astype(jnp.float32)
    ei = edge_index.astype(jnp.int32)
    rowp = jnp.concatenate([ei[0], jnp.zeros((EP - E,), jnp.int32)])
    colp = jnp.concatenate([ei[1], jnp.full((EP - E,), DUMMY, jnp.int32)])
    col_flat = colp.reshape(NW, EPT)
    row3 = rowp.reshape(TCH, CH)
    col3 = colp.reshape(TCH, CH)

    parts = _deg_kernel(col_flat)
    dis = _dis_call(parts)
    dis_col = dis.reshape(NP, 1)
    x_pad = jnp.concatenate([x, jnp.zeros((NP - N, D), jnp.float32)])
    y = _scale_call(x_pad, dis_col)
    p = _agg_kernel(y, row3, col3)
    h, mom = _mm_call(p, y, dis_col,
                      W.astype(jnp.float32),
                      b.astype(jnp.float32).reshape(1, D))
    return _bn_call(h, mom,
                    gamma.astype(jnp.float32).reshape(1, D),
                    beta.astype(jnp.float32).reshape(1, D))


# speedup vs baseline: 1.3685x; 1.0137x over previous
import functools

import jax
import jax.numpy as jnp
from jax import lax
from jax.experimental import pallas as pl
from jax.experimental.pallas import tpu as pltpu
from jax.experimental.pallas import tpu_sc as plsc

N = 10000
D = 128
E = 320000
EPS = 1e-5

NC = 2
NS = 16
NW = NC * NS
NP = 10240
EP = 327680
EPT = EP // NW
CH = 128
TCH = EP // CH
K0 = 128
K1 = 32
SL = 32
DUMMY = 10100
RB = 1280
G = NP // RB

_mesh = plsc.VectorSubcoreMesh(core_axis_name="c", subcore_axis_name="s")


@functools.partial(
    pl.kernel,
    out_type=jax.ShapeDtypeStruct((NW, NP), jnp.float32),
    mesh=_mesh,
    compiler_params=pltpu.CompilerParams(needs_layout_passes=False),
    scratch_types=[
        pltpu.VMEM((EPT,), jnp.int32),
        pltpu.VMEM((NP,), jnp.float32),
    ],
)
def _deg_kernel(col_hbm, out_hbm, col_v, deg_v):
    wid = lax.axis_index("s") * NC + lax.axis_index("c")
    pltpu.sync_copy(col_hbm.at[wid], col_v)
    zeros16 = jnp.zeros((16,), jnp.float32)

    def zbody(i, _):
        deg_v[pl.ds(i * 16, 16)] = zeros16
        return 0

    lax.fori_loop(0, NP // 16, zbody, 0)
    ones16 = jnp.ones((16,), jnp.float32)

    def hbody(i, _):
        idx = col_v[pl.ds(i * 16, 16)]
        plsc.addupdate_scatter(deg_v, [idx], ones16)
        return 0

    lax.fori_loop(0, EPT // 16, hbody, 0)
    pltpu.sync_copy(deg_v, out_hbm.at[wid])


@functools.partial(
    pl.kernel,
    out_type=jax.ShapeDtypeStruct((NC, NP, D), jnp.float32),
    mesh=_mesh,
    compiler_params=pltpu.CompilerParams(needs_layout_passes=False),
    scratch_types=[
        pltpu.VMEM((SL, CH), jnp.int32),
        pltpu.VMEM((SL, CH), jnp.int32),
        pltpu.VMEM((CH, D), jnp.float32),
        pltpu.VMEM((CH, D), jnp.float32),
        pltpu.VMEM_SHARED((NP, D), jnp.float32),
        pltpu.SemaphoreType.DMA,
        pltpu.SemaphoreType.DMA,
    ],
)
def _agg_kernel(y_hbm, row_hbm, col_hbm, out_hbm,
                row_v, col_v, buf0, buf1, acc, sem0, sem1):
    cid = lax.axis_index("c")
    sid = lax.axis_index("s")
    rpt = NP // NS

    zeros16 = jnp.zeros((16,), jnp.float32)

    def zbody(i, _):
        for k in range(D // 16):
            buf0[i, pl.ds(k * 16, 16)] = zeros16
        return 0

    lax.fori_loop(0, CH, zbody, 0)
    for k in range(rpt // CH):
        pltpu.sync_copy(buf0, acc.at[pl.ds(sid * rpt + k * CH, CH)])
    plsc.subcore_barrier()

    bufs = (buf0, buf1)
    sems = (sem0, sem1)

    def run_side(base, k):
        for slab in range(k // SL):
            cb = base + slab * SL
            pltpu.sync_copy(row_hbm.at[pl.ds(cb, SL)], row_v)
            pltpu.sync_copy(col_hbm.at[pl.ds(cb, SL)], col_v)
            pltpu.async_copy(y_hbm.at[row_v.at[0]], buf0, sem0)

            def body(i, _):
                for t in (0, 1):
                    j = 2 * i + t
                    pltpu.make_async_copy(y_hbm.at[row_v.at[j]], bufs[t],
                                          sems[t]).wait()

                    @pl.when(j + 1 < SL)
                    def _start_next():
                        pltpu.async_copy(y_hbm.at[row_v.at[j + 1]],
                                         bufs[1 - t], sems[1 - t])

                    pltpu.sync_copy(bufs[t], acc.at[col_v.at[j]], add=True)
                return 0

            lax.fori_loop(0, SL // 2, body, 0)

    @pl.when(cid == 0)
    def _side0():
        run_side(sid * K0, K0)

    @pl.when(cid == 1)
    def _side1():
        run_side(NS * K0 + sid * K1, K1)

    plsc.subcore_barrier()
    pltpu.sync_copy(acc.at[pl.ds(sid * rpt, rpt)],
                    out_hbm.at[cid, pl.ds(sid * rpt, rpt)])


def _dis_body(part_ref, o_ref):
    s = jnp.sum(part_ref[...], axis=0) + 1.0
    o_ref[...] = lax.rsqrt(s)


_dis_call = pl.pallas_call(
    _dis_body,
    out_shape=jax.ShapeDtypeStruct((NP,), jnp.float32),
)


def _scale_body(x_ref, dis_ref, o_ref):
    o_ref[...] = x_ref[...] * dis_ref[...]


_scale_call = pl.pallas_call(
    _scale_body,
    grid=(G,),
    in_specs=[
        pl.BlockSpec((RB, D), lambda i: (i, 0)),
        pl.BlockSpec((RB, 1), lambda i: (i, 0)),
    ],
    out_specs=pl.BlockSpec((RB, D), lambda i: (i, 0)),
    out_shape=jax.ShapeDtypeStruct((NP, D), jnp.float32),
)


def _mm_body(p_ref, y_ref, dis_ref, w_ref, b_ref, h_ref, mom_ref):
    i = pl.program_id(0)
    z = (p_ref[0] + p_ref[1] + y_ref[...]) * dis_ref[...]
    h = jnp.dot(z, w_ref[...], preferred_element_type=jnp.float32) + b_ref[...]
    h_ref[...] = h
    rows = i * RB + lax.broadcasted_iota(jnp.int32, (RB, 1), 0)
    hm = jnp.where(rows < N, h, 0.0)
    s1 = jnp.sum(hm, axis=0, keepdims=True)
    s2 = jnp.sum(hm * hm, axis=0, keepdims=True)
    mom_ref[...] = jnp.concatenate([s1, s2], axis=0).reshape(1, 2, D)


_mm_call = pl.pallas_call(
    _mm_body,
    grid=(G,),
    in_specs=[
        pl.BlockSpec((NC, RB, D), lambda i: (0, i, 0)),
        pl.BlockSpec((RB, D), lambda i: (i, 0)),
        pl.BlockSpec((RB, 1), lambda i: (i, 0)),
        pl.BlockSpec((D, D), lambda i: (0, 0)),
        pl.BlockSpec((1, D), lambda i: (0, 0)),
    ],
    out_specs=[
        pl.BlockSpec((RB, D), lambda i: (i, 0)),
        pl.BlockSpec((1, 2, D), lambda i: (i, 0, 0)),
    ],
    out_shape=[
        jax.ShapeDtypeStruct((NP, D), jnp.float32),
        jax.ShapeDtypeStruct((G, 2, D), jnp.float32),
    ],
)


def _bn_body(h_ref, mom_ref, g_ref, bt_ref, o_ref):
    m = jnp.sum(mom_ref[...], axis=0)
    mean = m[0:1] * (1.0 / N)
    var = m[1:2] * (1.0 / N) - mean * mean
    scale = g_ref[...] * lax.rsqrt(var + EPS)
    shift = bt_ref[...] - mean * scale
    o_ref[...] = jnp.maximum(h_ref[...] * scale + shift, 0.0)


_BNB = 1000

_bn_call = pl.pallas_call(
    _bn_body,
    grid=(N // _BNB,),
    in_specs=[
        pl.BlockSpec((_BNB, D), lambda i: (i, 0)),
        pl.BlockSpec((G, 2, D), lambda i: (0, 0, 0)),
        pl.BlockSpec((1, D), lambda i: (0, 0)),
        pl.BlockSpec((1, D), lambda i: (0, 0)),
    ],
    out_specs=pl.BlockSpec((_BNB, D), lambda i: (i, 0)),
    out_shape=jax.ShapeDtypeStruct((N, D), jnp.float32),
)


def kernel(x, edge_index, W, b, gamma, beta):
    x = x.astype(jnp.float32)
    ei = edge_index.astype(jnp.int32)
    rowp = jnp.concatenate([ei[0], jnp.zeros((EP - E,), jnp.int32)])
    colp = jnp.concatenate([ei[1], jnp.full((EP - E,), DUMMY, jnp.int32)])
    col_flat = colp.reshape(NW, EPT)
    row3 = rowp.reshape(TCH, CH)
    col3 = colp.reshape(TCH, CH)

    parts = _deg_kernel(col_flat)
    dis = _dis_call(parts)
    dis_col = dis.reshape(NP, 1)
    x_pad = jnp.concatenate([x, jnp.zeros((NP - N, D), jnp.float32)])
    y = _scale_call(x_pad, dis_col)
    p = _agg_kernel(y, row3, col3)
    h, mom = _mm_call(p, y, dis_col,
                      W.astype(jnp.float32),
                      b.astype(jnp.float32).reshape(1, D))
    return _bn_call(h, mom,
                    gamma.astype(jnp.float32).reshape(1, D),
                    beta.astype(jnp.float32).reshape(1, D))


# speedup vs baseline: 1.4314x; 1.0460x over previous
import functools

import jax
import jax.numpy as jnp
from jax import lax
from jax.experimental import pallas as pl
from jax.experimental.pallas import tpu as pltpu
from jax.experimental.pallas import tpu_sc as plsc

N = 10000
D = 128
E = 320000
EPS = 1e-5

NC = 2
NS = 16
NW = NC * NS
NP = 10240
EP = 327680
EPT = EP // NW
CH = 128
TCH = EP // CH
K0 = 144
K1 = 16
SL0 = 48
SL1 = 16
SL = 48
DUMMY = 10100
RB = 1280
G = NP // RB

_mesh = plsc.VectorSubcoreMesh(core_axis_name="c", subcore_axis_name="s")


@functools.partial(
    pl.kernel,
    out_type=jax.ShapeDtypeStruct((NW, NP), jnp.float32),
    mesh=_mesh,
    compiler_params=pltpu.CompilerParams(needs_layout_passes=False),
    scratch_types=[
        pltpu.VMEM((EPT,), jnp.int32),
        pltpu.VMEM((NP,), jnp.float32),
    ],
)
def _deg_kernel(col_hbm, out_hbm, col_v, deg_v):
    wid = lax.axis_index("s") * NC + lax.axis_index("c")
    pltpu.sync_copy(col_hbm.at[wid], col_v)
    zeros16 = jnp.zeros((16,), jnp.float32)

    def zbody(i, _):
        deg_v[pl.ds(i * 16, 16)] = zeros16
        return 0

    lax.fori_loop(0, NP // 16, zbody, 0)
    ones16 = jnp.ones((16,), jnp.float32)

    def hbody(i, _):
        idx = col_v[pl.ds(i * 16, 16)]
        plsc.addupdate_scatter(deg_v, [idx], ones16)
        return 0

    lax.fori_loop(0, EPT // 16, hbody, 0)
    pltpu.sync_copy(deg_v, out_hbm.at[wid])


@functools.partial(
    pl.kernel,
    out_type=jax.ShapeDtypeStruct((NC, NP, D), jnp.float32),
    mesh=_mesh,
    compiler_params=pltpu.CompilerParams(needs_layout_passes=False),
    scratch_types=[
        pltpu.VMEM((SL, CH), jnp.int32),
        pltpu.VMEM((SL, CH), jnp.int32),
        pltpu.VMEM((CH, D), jnp.float32),
        pltpu.VMEM((CH, D), jnp.float32),
        pltpu.VMEM_SHARED((NP, D), jnp.float32),
        pltpu.SemaphoreType.DMA,
        pltpu.SemaphoreType.DMA,
    ],
)
def _agg_kernel(y_hbm, row_hbm, col_hbm, out_hbm,
                row_v, col_v, buf0, buf1, acc, sem0, sem1):
    cid = lax.axis_index("c")
    sid = lax.axis_index("s")
    rpt = NP // NS

    zeros16 = jnp.zeros((16,), jnp.float32)

    def zbody(i, _):
        for k in range(D // 16):
            buf0[i, pl.ds(k * 16, 16)] = zeros16
        return 0

    lax.fori_loop(0, CH, zbody, 0)
    for k in range(rpt // CH):
        pltpu.sync_copy(buf0, acc.at[pl.ds(sid * rpt + k * CH, CH)])
    plsc.subcore_barrier()

    bufs = (buf0, buf1)
    sems = (sem0, sem1)

    def run_side(base, k, sl):
        for slab in range(k // sl):
            cb = base + slab * sl
            pltpu.sync_copy(row_hbm.at[pl.ds(cb, sl)],
                            row_v.at[pl.ds(0, sl)])
            pltpu.sync_copy(col_hbm.at[pl.ds(cb, sl)],
                            col_v.at[pl.ds(0, sl)])
            pltpu.async_copy(y_hbm.at[row_v.at[0]], buf0, sem0)

            def body(i, _):
                for t in (0, 1):
                    j = 2 * i + t
                    pltpu.make_async_copy(y_hbm.at[row_v.at[j]], bufs[t],
                                          sems[t]).wait()

                    @pl.when(j + 1 < sl)
                    def _start_next():
                        pltpu.async_copy(y_hbm.at[row_v.at[j + 1]],
                                         bufs[1 - t], sems[1 - t])

                    pltpu.sync_copy(bufs[t], acc.at[col_v.at[j]], add=True)
                return 0

            lax.fori_loop(0, sl // 2, body, 0)

    @pl.when(cid == 0)
    def _side0():
        run_side(sid * K0, K0, SL0)

    @pl.when(cid == 1)
    def _side1():
        run_side(NS * K0 + sid * K1, K1, SL1)

    plsc.subcore_barrier()
    pltpu.sync_copy(acc.at[pl.ds(sid * rpt, rpt)],
                    out_hbm.at[cid, pl.ds(sid * rpt, rpt)])


def _dis_body(part_ref, o_ref):
    s = jnp.sum(part_ref[...], axis=0) + 1.0
    o_ref[...] = lax.rsqrt(s)


_dis_call = pl.pallas_call(
    _dis_body,
    out_shape=jax.ShapeDtypeStruct((NP,), jnp.float32),
)


def _scale_body(x_ref, dis_ref, o_ref):
    o_ref[...] = x_ref[...] * dis_ref[...]


_scale_call = pl.pallas_call(
    _scale_body,
    grid=(G,),
    in_specs=[
        pl.BlockSpec((RB, D), lambda i: (i, 0)),
        pl.BlockSpec((RB, 1), lambda i: (i, 0)),
    ],
    out_specs=pl.BlockSpec((RB, D), lambda i: (i, 0)),
    out_shape=jax.ShapeDtypeStruct((NP, D), jnp.float32),
)


def _mm_body(p_ref, y_ref, dis_ref, w_ref, b_ref, h_ref, mom_ref):
    i = pl.program_id(0)
    z = (p_ref[0] + p_ref[1] + y_ref[...]) * dis_ref[...]
    h = jnp.dot(z, w_ref[...], preferred_element_type=jnp.float32) + b_ref[...]
    h_ref[...] = h
    rows = i * RB + lax.broadcasted_iota(jnp.int32, (RB, 1), 0)
    hm = jnp.where(rows < N, h, 0.0)
    s1 = jnp.sum(hm, axis=0, keepdims=True)
    s2 = jnp.sum(hm * hm, axis=0, keepdims=True)
    mom_ref[...] = jnp.concatenate([s1, s2], axis=0).reshape(1, 2, D)


_mm_call = pl.pallas_call(
    _mm_body,
    grid=(G,),
    in_specs=[
        pl.BlockSpec((NC, RB, D), lambda i: (0, i, 0)),
        pl.BlockSpec((RB, D), lambda i: (i, 0)),
        pl.BlockSpec((RB, 1), lambda i: (i, 0)),
        pl.BlockSpec((D, D), lambda i: (0, 0)),
        pl.BlockSpec((1, D), lambda i: (0, 0)),
    ],
    out_specs=[
        pl.BlockSpec((RB, D), lambda i: (i, 0)),
        pl.BlockSpec((1, 2, D), lambda i: (i, 0, 0)),
    ],
    out_shape=[
        jax.ShapeDtypeStruct((NP, D), jnp.float32),
        jax.ShapeDtypeStruct((G, 2, D), jnp.float32),
    ],
)


def _bn_body(h_ref, mom_ref, g_ref, bt_ref, o_ref):
    m = jnp.sum(mom_ref[...], axis=0)
    mean = m[0:1] * (1.0 / N)
    var = m[1:2] * (1.0 / N) - mean * mean
    scale = g_ref[...] * lax.rsqrt(var + EPS)
    shift = bt_ref[...] - mean * scale
    o_ref[...] = jnp.maximum(h_ref[...] * scale + shift, 0.0)


_BNB = 1000

_bn_call = pl.pallas_call(
    _bn_body,
    grid=(N // _BNB,),
    in_specs=[
        pl.BlockSpec((_BNB, D), lambda i: (i, 0)),
        pl.BlockSpec((G, 2, D), lambda i: (0, 0, 0)),
        pl.BlockSpec((1, D), lambda i: (0, 0)),
        pl.BlockSpec((1, D), lambda i: (0, 0)),
    ],
    out_specs=pl.BlockSpec((_BNB, D), lambda i: (i, 0)),
    out_shape=jax.ShapeDtypeStruct((N, D), jnp.float32),
)


def kernel(x, edge_index, W, b, gamma, beta):
    x = x.astype(jnp.float32)
    ei = edge_index.astype(jnp.int32)
    rowp = jnp.concatenate([ei[0], jnp.zeros((EP - E,), jnp.int32)])
    colp = jnp.concatenate([ei[1], jnp.full((EP - E,), DUMMY, jnp.int32)])
    col_flat = colp.reshape(NW, EPT)
    row3 = rowp.reshape(TCH, CH)
    col3 = colp.reshape(TCH, CH)

    parts = _deg_kernel(col_flat)
    dis = _dis_call(parts)
    dis_col = dis.reshape(NP, 1)
    x_pad = jnp.concatenate([x, jnp.zeros((NP - N, D), jnp.float32)])
    y = _scale_call(x_pad, dis_col)
    p = _agg_kernel(y, row3, col3)
    h, mom = _mm_call(p, y, dis_col,
                      W.astype(jnp.float32),
                      b.astype(jnp.float32).reshape(1, D))
    return _bn_call(h, mom,
                    gamma.astype(jnp.float32).reshape(1, D),
                    beta.astype(jnp.float32).reshape(1, D))
